# Initial kernel scaffold; baseline (speedup 1.0000x reference)
import jax
import jax.numpy as jnp
from jax.experimental import pallas as pl


def kernel(relation):
    raise NotImplementedError("write your pallas kernel here")


# speedup vs baseline: 18.0468x; 18.0468x over previous
import jax
import jax.numpy as jnp
from jax.experimental import pallas as pl

_K = 128
_BR = 256
_HI = 0x7F800000


def _topk_mask_body(x_ref, o_ref):
    x = x_ref[...]
    xb = jax.lax.bitcast_convert_type(x, jnp.int32)
    rows = x.shape[0]
    lo = jnp.zeros((rows, 1), jnp.int32)
    hi = jnp.full((rows, 1), _HI, jnp.int32)

    def body(_, carry):
        lo, hi = carry
        mid = lo + ((hi - lo) >> 1)
        cnt = jnp.sum((xb >= mid).astype(jnp.int32), axis=1, keepdims=True)
        pred = cnt >= _K
        lo = jnp.where(pred, mid, lo)
        hi = jnp.where(pred, hi, mid)
        return lo, hi

    lo, hi = jax.lax.fori_loop(0, 31, body, (lo, hi))
    o_ref[...] = jnp.where(xb >= lo, x, 0.0)


def kernel(relation):
    n, m = relation.shape
    return pl.pallas_call(
        _topk_mask_body,
        grid=(n // _BR,),
        in_specs=[pl.BlockSpec((_BR, m), lambda i: (i, 0))],
        out_specs=pl.BlockSpec((_BR, m), lambda i: (i, 0)),
        out_shape=jax.ShapeDtypeStruct((n, m), jnp.float32),
    )(relation)
